# baseline (speedup 1.0000x reference)
import functools

import jax
import jax.numpy as jnp
from jax.experimental import pallas as pl
from jax.experimental.pallas import tpu as pltpu

N_ASSETS = 64
N_REGIMES = 8
HIDDEN = 256


def _chain(xt, w1, b1, w2, b2, wh, bh, rows):
    reg = xt[:, HIDDEN - 1:HIDDEN].astype(jnp.int32)
    h = jnp.dot(xt, w1, preferred_element_type=jnp.float32)
    h = jnp.maximum(h + b1, 0.0)
    f = jnp.dot(h, w2, preferred_element_type=jnp.float32)
    f = jnp.maximum(f + b2, 0.0)
    oa = jnp.dot(f, wh, preferred_element_type=jnp.float32)
    oa = oa + bh
    wide = N_REGIMES * N_ASSETS
    col = jax.lax.broadcasted_iota(jnp.int32, (rows, wide), 1)
    sel = jnp.where(col // N_ASSETS == reg, oa, 0.0)
    fold_c = jax.lax.broadcasted_iota(jnp.int32, (wide, N_ASSETS), 0)
    fold_a = jax.lax.broadcasted_iota(jnp.int32, (wide, N_ASSETS), 1)
    fold = jnp.where(fold_c % N_ASSETS == fold_a, 0.1, 0.0)
    return jnp.dot(sel, fold, preferred_element_type=jnp.float32)


def _body(x_ref, w1_ref, b1_ref, w2_ref, b2_ref, wh_ref, bh_ref, ls_ref,
          mean_ref, std_ref, *, tile_rows):
    quarter = tile_rows // 16
    args = (w1_ref[...], b1_ref[...], w2_ref[...], b2_ref[...],
            wh_ref[...], bh_ref[...])
    for q in range(16):
        sl = slice(q * quarter, (q + 1) * quarter)
        mean_ref[sl, :] = _chain(x_ref[sl, :], *args, quarter)
    std = jnp.clip(jnp.exp(ls_ref[...]), 1e-3, 1.0)
    std_ref[...] = jnp.broadcast_to(std, (tile_rows, N_ASSETS))


def kernel(x, W1, b1, W2, b2, Wh, bh, log_std):
    batch, in_dim = x.shape
    tile_rows = 8192
    grid = (batch // tile_rows,)

    wh_all = jnp.transpose(Wh, (1, 0, 2)).reshape(HIDDEN, N_REGIMES * N_ASSETS)
    bh_all = bh.reshape(1, N_REGIMES * N_ASSETS)
    b1r = b1.reshape(1, HIDDEN)
    b2r = b2.reshape(1, HIDDEN)
    lsr = log_std.reshape(1, N_ASSETS)

    const = lambda *_: (0, 0)
    mean, std = pl.pallas_call(
        functools.partial(_body, tile_rows=tile_rows),
        grid=grid,
        in_specs=[
            pl.BlockSpec((tile_rows, in_dim), lambda i: (i, 0)),
            pl.BlockSpec((in_dim, HIDDEN), const),
            pl.BlockSpec((1, HIDDEN), const),
            pl.BlockSpec((HIDDEN, HIDDEN), const),
            pl.BlockSpec((1, HIDDEN), const),
            pl.BlockSpec((HIDDEN, N_REGIMES * N_ASSETS), const),
            pl.BlockSpec((1, N_REGIMES * N_ASSETS), const),
            pl.BlockSpec((1, N_ASSETS), const),
        ],
        out_specs=[
            pl.BlockSpec((tile_rows, N_ASSETS), lambda i: (i, 0)),
            pl.BlockSpec((tile_rows, N_ASSETS), lambda i: (i, 0)),
        ],
        out_shape=[
            jax.ShapeDtypeStruct((batch, N_ASSETS), jnp.float32),
            jax.ShapeDtypeStruct((batch, N_ASSETS), jnp.float32),
        ],
        compiler_params=pltpu.CompilerParams(
            dimension_semantics=("arbitrary",),
        ),
    )(x, W1, b1r, W2, b2r, wh_all, bh_all, lsr)
    return (mean, std)


# speedup vs baseline: 1.0487x; 1.0487x over previous
import functools

import jax
import jax.numpy as jnp
from jax.experimental import pallas as pl
from jax.experimental.pallas import tpu as pltpu

N_ASSETS = 64
N_REGIMES = 8
HIDDEN = 256


def _chain(xt, w1, b1, w2, b2, wh, bh, rows):
    reg = xt[:, HIDDEN - 1:HIDDEN].astype(jnp.int32)
    h = jnp.dot(xt, w1, preferred_element_type=jnp.float32)
    h = jnp.maximum(h + b1, 0.0)
    f = jnp.dot(h, w2, preferred_element_type=jnp.float32)
    f = jnp.maximum(f + b2, 0.0)
    oa = jnp.dot(f, wh, preferred_element_type=jnp.float32)
    oa = oa + bh
    wide = N_REGIMES * N_ASSETS
    col = jax.lax.broadcasted_iota(jnp.int32, (rows, wide), 1)
    sel = jnp.where(col // N_ASSETS == reg, oa, 0.0)
    fold_c = jax.lax.broadcasted_iota(jnp.int32, (wide, N_ASSETS), 0)
    fold_a = jax.lax.broadcasted_iota(jnp.int32, (wide, N_ASSETS), 1)
    fold = jnp.where(fold_c % N_ASSETS == fold_a, 0.1, 0.0)
    return jnp.dot(sel, fold, preferred_element_type=jnp.float32)


def _body(x_ref, w1_ref, b1_ref, w2_ref, b2_ref, wh_ref, bh_ref, ls_ref,
          mean_ref, std_ref, *, tile_rows):
    sub_rows = tile_rows // 8
    args = (w1_ref[...], b1_ref[...], w2_ref[...], b2_ref[...],
            wh_ref[...], bh_ref[...])
    for q in range(8):
        sl = slice(q * sub_rows, (q + 1) * sub_rows)
        mean_ref[sl, :] = _chain(x_ref[sl, :], *args, sub_rows)
    std = jnp.clip(jnp.exp(ls_ref[...]), 1e-3, 1.0)
    std_ref[...] = jnp.broadcast_to(std, (tile_rows, N_ASSETS))


def kernel(x, W1, b1, W2, b2, Wh, bh, log_std):
    batch, in_dim = x.shape
    tile_rows = 4096
    grid = (batch // tile_rows,)

    wh_all = jnp.transpose(Wh, (1, 0, 2)).reshape(HIDDEN, N_REGIMES * N_ASSETS)
    bh_all = bh.reshape(1, N_REGIMES * N_ASSETS)
    b1r = b1.reshape(1, HIDDEN)
    b2r = b2.reshape(1, HIDDEN)
    lsr = log_std.reshape(1, N_ASSETS)

    const = lambda *_: (0, 0)
    mean, std = pl.pallas_call(
        functools.partial(_body, tile_rows=tile_rows),
        grid=grid,
        in_specs=[
            pl.BlockSpec((tile_rows, in_dim), lambda i: (i, 0)),
            pl.BlockSpec((in_dim, HIDDEN), const),
            pl.BlockSpec((1, HIDDEN), const),
            pl.BlockSpec((HIDDEN, HIDDEN), const),
            pl.BlockSpec((1, HIDDEN), const),
            pl.BlockSpec((HIDDEN, N_REGIMES * N_ASSETS), const),
            pl.BlockSpec((1, N_REGIMES * N_ASSETS), const),
            pl.BlockSpec((1, N_ASSETS), const),
        ],
        out_specs=[
            pl.BlockSpec((tile_rows, N_ASSETS), lambda i: (i, 0)),
            pl.BlockSpec((tile_rows, N_ASSETS), lambda i: (i, 0)),
        ],
        out_shape=[
            jax.ShapeDtypeStruct((batch, N_ASSETS), jnp.float32),
            jax.ShapeDtypeStruct((batch, N_ASSETS), jnp.float32),
        ],
        compiler_params=pltpu.CompilerParams(
            dimension_semantics=("arbitrary",),
        ),
    )(x, W1, b1r, W2, b2r, wh_all, bh_all, lsr)
    return (mean, std)
